# baseline (speedup 1.0000x reference)
import functools

import numpy as np
import jax
import jax.numpy as jnp
from jax import lax
from jax.experimental import pallas as pl
from jax.experimental.pallas import tpu as pltpu
from jax.experimental.pallas import tpu_sc as plsc

D = 8
NBLADES = 256
BATCH = 1024
NTILES = 32
COLS = BATCH // NTILES


def _popcount(x: int) -> int:
    return bin(x).count("1")


_ORDER = sorted(
    range(NBLADES),
    key=lambda m: (_popcount(m), tuple(i for i in range(D) if (m >> i) & 1)),
)

_I2 = np.eye(2)
_X = np.array([[0.0, 1.0], [1.0, 0.0]])
_Z = np.array([[1.0, 0.0], [0.0, -1.0]])
_C = _X @ _Z
_FACT = {0: _I2, 1: _X, 2: _Z, 3: _C}
_GENS = [
    (0, 0, 0, 0, 1), (0, 0, 0, 0, 2), (0, 0, 0, 3, 3), (0, 0, 3, 1, 3),
    (0, 1, 3, 2, 3), (0, 2, 3, 2, 3), (0, 3, 0, 2, 3), (3, 3, 1, 1, 3),
]


def _kron5(u):
    m = _FACT[u[0]]
    for k in u[1:]:
        m = np.kron(m, _FACT[k])
    return m


_G = [_kron5(u) for u in _GENS]
for _i, _g in enumerate(_G):
    assert np.allclose(_g @ _g, (1.0 if _i < 7 else -1.0) * np.eye(32))
    for _j in range(_i):
        assert np.allclose(_g @ _G[_j], -_G[_j] @ _g)

_R = [None] * NBLADES
_R[0] = np.eye(32)
for _m in range(1, NBLADES):
    _h = _m.bit_length() - 1
    _R[_m] = _R[_m ^ (1 << _h)] @ _G[_h]

_SEL = [0, 1, 2, 3, 8, 9, 10, 11]
_F = np.zeros((256, 256))
for _c in range(NBLADES):
    _F[:, _c] = _R[_c][:, _SEL].reshape(-1)
_FINV_MASK = np.linalg.inv(_F)
_FINV_MASK = np.round(_FINV_MASK * 8.0) / 8.0

_EA = np.zeros((256, 1024), dtype=np.float32)
_EB = np.zeros((256, 256), dtype=np.float32)
_FINV = np.zeros((256, 256), dtype=np.float32)
for _ib in range(NBLADES):
    _m = _ORDER[_ib]
    _EA[_ib, :] = _R[_m].reshape(-1)
    _EB[_ib, :] = _R[_m][:, _SEL].reshape(-1)
    _FINV[_ib, :] = _FINV_MASK[_m, :]

_mesh = plsc.VectorSubcoreMesh(core_axis_name="c", subcore_axis_name="s")


@functools.partial(
    pl.kernel,
    out_type=jax.ShapeDtypeStruct((NBLADES, BATCH), jnp.float32),
    mesh=_mesh,
    scratch_types=[
        pltpu.VMEM((1024, COLS), jnp.float32),
        pltpu.VMEM((256, COLS), jnp.float32),
        pltpu.VMEM((256, COLS), jnp.float32),
        pltpu.SemaphoreType.DMA,
    ],
    compiler_params=pltpu.CompilerParams(
        use_tc_tiling_on_sc=False, needs_layout_passes=False,
        skip_device_barrier=True),
)
def _gp_sc(af_hbm, bs_hbm, out_hbm, af_v, bs_v, o_v, sem):
    wid = lax.axis_index("s") * 2 + lax.axis_index("c")
    c0 = wid * COLS

    cp_b = pltpu.async_copy(bs_hbm.at[:, pl.ds(c0, COLS)], bs_v, sem)
    cp_a0 = pltpu.async_copy(
        af_hbm.at[:, pl.ds(c0, 16)], af_v.at[:, pl.ds(0, 16)], sem)
    cp_a1 = pltpu.async_copy(
        af_hbm.at[:, pl.ds(c0 + 16, 16)], af_v.at[:, pl.ds(16, 16)], sem)
    cp_b.wait()
    cp_a0.wait()

    for g in range(2):
        col = g * 16
        if g == 1:
            cp_a1.wait()

        def ib_body(ib, carry, col=col):
            i0 = ib * 4

            def j_body(j, accs, i0=i0, col=col):
                avs = [af_v[(i0 + u) * 32 + j, pl.ds(col, 16)]
                       for u in range(4)]
                bvs = [bs_v[j * 8 + s, pl.ds(col, 16)] for s in range(8)]
                new = list(accs)
                for u in range(4):
                    for s in range(8):
                        new[u * 8 + s] = new[u * 8 + s] + avs[u] * bvs[s]
                return tuple(new)

            accs0 = tuple(jnp.zeros((16,), jnp.float32) for _ in range(32))
            accs = lax.fori_loop(0, 32, j_body, accs0)
            for u in range(4):
                for s in range(8):
                    o_v[(i0 + u) * 8 + s, pl.ds(col, 16)] = accs[u * 8 + s]
            return carry

        lax.fori_loop(0, 8, ib_body, 0)

    pltpu.sync_copy(o_v, out_hbm.at[:, pl.ds(c0, COLS)])


def kernel(a, b):
    af = jnp.einsum("nc,cs->sn", a, _EA, preferred_element_type=jnp.float32)
    bs = jnp.einsum("nc,cs->sn", b, _EB, preferred_element_type=jnp.float32)
    rsel = _gp_sc(af, bs)
    return jnp.einsum("sn,cs->nc", rsel, _FINV,
                      preferred_element_type=jnp.float32)


# speedup vs baseline: 1.2098x; 1.2098x over previous
import functools

import numpy as np
import jax
import jax.numpy as jnp
from jax import lax
from jax.experimental import pallas as pl
from jax.experimental.pallas import tpu as pltpu
from jax.experimental.pallas import tpu_sc as plsc

D = 8
NBLADES = 256
BATCH = 1024
NTILES = 32
COLS = BATCH // NTILES


def _popcount(x: int) -> int:
    return bin(x).count("1")


_ORDER = sorted(
    range(NBLADES),
    key=lambda m: (_popcount(m), tuple(i for i in range(D) if (m >> i) & 1)),
)

_I2 = np.eye(2)
_X = np.array([[0.0, 1.0], [1.0, 0.0]])
_Z = np.array([[1.0, 0.0], [0.0, -1.0]])
_C = _X @ _Z
_FACT = {0: _I2, 1: _X, 2: _Z, 3: _C}
_GENS = [
    (0, 0, 0, 0, 1), (0, 0, 0, 0, 2), (0, 0, 0, 3, 3), (0, 0, 3, 1, 3),
    (0, 1, 3, 2, 3), (0, 2, 3, 2, 3), (0, 3, 0, 2, 3), (3, 3, 1, 1, 3),
]


def _kron5(u):
    m = _FACT[u[0]]
    for k in u[1:]:
        m = np.kron(m, _FACT[k])
    return m


_G = [_kron5(u) for u in _GENS]
for _i, _g in enumerate(_G):
    assert np.allclose(_g @ _g, (1.0 if _i < 7 else -1.0) * np.eye(32))
    for _j in range(_i):
        assert np.allclose(_g @ _G[_j], -_G[_j] @ _g)

_R = [None] * NBLADES
_R[0] = np.eye(32)
for _m in range(1, NBLADES):
    _h = _m.bit_length() - 1
    _R[_m] = _R[_m ^ (1 << _h)] @ _G[_h]

_SEL = [0, 1, 2, 3, 8, 9, 10, 11]
_F = np.zeros((256, 256))
for _c in range(NBLADES):
    _F[:, _c] = _R[_c][:, _SEL].reshape(-1)
_FINV_MASK = np.linalg.inv(_F)
_FINV_MASK = np.round(_FINV_MASK * 8.0) / 8.0

_EA = np.zeros((256, 1024), dtype=np.float32)
_EB = np.zeros((256, 256), dtype=np.float32)
_FINV = np.zeros((256, 256), dtype=np.float32)
for _ib in range(NBLADES):
    _m = _ORDER[_ib]
    _EA[_ib, :] = _R[_m].reshape(-1)
    _EB[_ib, :] = _R[_m][:, _SEL].reshape(-1)
    _FINV[_ib, :] = _FINV_MASK[_m, :]

_mesh = plsc.VectorSubcoreMesh(core_axis_name="c", subcore_axis_name="s")


@functools.partial(
    pl.kernel,
    out_type=jax.ShapeDtypeStruct((NBLADES, BATCH), jnp.float32),
    mesh=_mesh,
    scratch_types=[
        pltpu.VMEM((1024, COLS), jnp.float32),
        pltpu.VMEM((256, COLS), jnp.float32),
        pltpu.VMEM((256, COLS), jnp.float32),
        pltpu.SemaphoreType.DMA,
    ],
    compiler_params=pltpu.CompilerParams(
        use_tc_tiling_on_sc=False, needs_layout_passes=False,
        skip_device_barrier=True),
)
def _gp_sc(af_hbm, bs_hbm, out_hbm, af_v, bs_v, o_v, sem):
    wid = lax.axis_index("s") * 2 + lax.axis_index("c")
    c0 = wid * COLS

    cp_a = pltpu.async_copy(af_hbm.at[:, pl.ds(c0, COLS)], af_v, sem)
    cp_b = pltpu.async_copy(bs_hbm.at[:, pl.ds(c0, COLS)], bs_v, sem)
    cp_a.wait()
    cp_b.wait()

    for g in range(2):
        col = g * 16

        def ib_body(ib, carry, col=col):
            i0 = ib * 4

            def j_body(j, accs, i0=i0, col=col):
                avs = [af_v[(i0 + u) * 32 + j, pl.ds(col, 16)]
                       for u in range(4)]
                bvs = [bs_v[j * 8 + s, pl.ds(col, 16)] for s in range(8)]
                new = list(accs)
                for u in range(4):
                    for s in range(8):
                        new[u * 8 + s] = new[u * 8 + s] + avs[u] * bvs[s]
                return tuple(new)

            accs0 = tuple(jnp.zeros((16,), jnp.float32) for _ in range(32))
            accs = lax.fori_loop(0, 32, j_body, accs0)
            for u in range(4):
                for s in range(8):
                    o_v[(i0 + u) * 8 + s, pl.ds(col, 16)] = accs[u * 8 + s]
            return carry

        lax.fori_loop(0, 8, ib_body, 0)

    pltpu.sync_copy(o_v, out_hbm.at[:, pl.ds(c0, COLS)])


def kernel(a, b):
    af = jnp.einsum("nc,cs->sn", a, _EA, preferred_element_type=jnp.float32)
    bs = jnp.einsum("nc,cs->sn", b, _EB, preferred_element_type=jnp.float32)
    rsel = _gp_sc(af, bs)
    return jnp.einsum("sn,cs->nc", rsel, _FINV,
                      preferred_element_type=jnp.float32)
